# baseline (speedup 1.0000x reference)
import jax
import jax.numpy as jnp
from jax.experimental import pallas as pl

T, C = 2048, 1024
H, HS = 16, 64
NR, NT = 6, 2
E = NR + NT
FF = 2048

_TB1 = 512
_QB = 1024
_TB3 = 256
_FFC = 512
_CC = 256


def _ln(x, g, b):
    m = jnp.mean(x, axis=-1, keepdims=True)
    v = jnp.var(x, axis=-1, keepdims=True)
    return (x - m) / jnp.sqrt(v + 1e-5) * g + b


def _dot(a, b):
    return jnp.dot(a, b, preferred_element_type=jnp.float32)


def _k1(x_ref, wr_ref, wk_ref, wv_ref, r_ref, k_ref, v_ref):
    xln = x_ref[...]
    r_ref[...] = _dot(xln, wr_ref[...])
    k_ref[...] = _dot(xln, wk_ref[...])
    v_ref[...] = _dot(xln, wv_ref[...])


def _qkv(x_ln, Wr, Wk, Wv):
    n = T // _TB1
    blk = pl.BlockSpec((_TB1, C), lambda i: (i, 0))
    wspec = pl.BlockSpec((C, C), lambda i: (0, 0))
    return pl.pallas_call(
        _k1,
        grid=(n,),
        in_specs=[blk, wspec, wspec, wspec],
        out_specs=[blk, blk, blk],
        out_shape=[jax.ShapeDtypeStruct((T, C), jnp.float32)] * 3,
    )(x_ln, Wr, Wk, Wv)


def _k2a(r_ref, k_ref, o_ref):
    qi = pl.program_id(1)
    s = jax.lax.dot_general(
        r_ref[0], k_ref[0], (((1,), (1,)), ((), ())),
        preferred_element_type=jnp.float32) / 8.0
    rows = qi * _QB + jax.lax.broadcasted_iota(jnp.int32, (_QB, T), 0)
    cols = jax.lax.broadcasted_iota(jnp.int32, (_QB, T), 1)
    s = jnp.where(cols <= rows, s, -1e9)
    m = jnp.max(s, axis=-1, keepdims=True)
    o_ref[0] = jnp.exp(s - m)


def _scores(rh, kh):
    qspec = pl.BlockSpec((1, _QB, HS), lambda h, q: (h, q, 0))
    kspec = pl.BlockSpec((1, T, HS), lambda h, q: (h, 0, 0))
    return pl.pallas_call(
        _k2a,
        grid=(H, T // _QB),
        in_specs=[qspec, kspec],
        out_specs=pl.BlockSpec((1, _QB, T), lambda h, q: (h, q, 0)),
        out_shape=jax.ShapeDtypeStruct((H, T, T), jnp.float32),
    )(rh, kh)


def _k2b(pu_ref, z_ref, v_ref, o_ref):
    o_ref[0] = _dot(pu_ref[0] / z_ref[0], v_ref[0])


def _pv(pu, z, vh):
    return pl.pallas_call(
        _k2b,
        grid=(H, T // _QB),
        in_specs=[pl.BlockSpec((1, _QB, T), lambda h, q: (h, q, 0)),
                  pl.BlockSpec((1, _QB, 1), lambda h, q: (h, q, 0)),
                  pl.BlockSpec((1, T, HS), lambda h, q: (h, 0, 0))],
        out_specs=pl.BlockSpec((1, _QB, HS), lambda h, q: (h, q, 0)),
        out_shape=jax.ShapeDtypeStruct((H, T, HS), jnp.float32),
    )(pu, z, vh)


def _k3a(x_ref, mix_ref, wo_ref, o_ref):
    o_ref[...] = x_ref[...] + _dot(mix_ref[...], wo_ref[...])


def _resid(x, mix, Wo):
    n = T // _TB3
    blk = pl.BlockSpec((_TB3, C), lambda i: (i, 0))
    wspec = pl.BlockSpec((C, C), lambda i: (0, 0))
    return pl.pallas_call(
        _k3a,
        grid=(n,),
        in_specs=[blk, blk, wspec],
        out_specs=blk,
        out_shape=jax.ShapeDtypeStruct((T, C), jnp.float32),
    )(x, mix, Wo)


def _k3b(h_ref, mix_ref, wbh_ref, wbs_ref, bb_ref,
         cw_ref, cb_ref, wd_ref, wa_ref, cs_ref,
         pre_ref, gates_ref, win_ref, cost_ref, diff_ref, aff_ref):
    h = h_ref[...]
    mix = mix_ref[...]
    pre_ref[...] = jnp.maximum(
        _dot(h, wbh_ref[...]) + _dot(mix, wbs_ref[...]) + bb_ref[...], 0.0)
    conf = jax.nn.sigmoid(_dot(h, cw_ref[...]) + cb_ref[...])
    diff = jax.nn.sigmoid(_dot(h, wd_ref[...]))
    aff = _dot(h, wa_ref[...])
    bids = conf * cs_ref[...] * diff + 0.1 * aff
    idx = jax.lax.broadcasted_iota(jnp.int32, bids.shape, 1)
    m1 = jnp.max(bids, axis=-1, keepdims=True)
    a1 = jnp.argmax(bids, axis=-1, keepdims=True).astype(jnp.int32)
    masked = jnp.where(idx == a1, -jnp.inf, bids)
    m2 = jnp.max(masked, axis=-1, keepdims=True)
    a2 = jnp.argmax(masked, axis=-1, keepdims=True).astype(jnp.int32)
    e2 = jnp.exp(m2 - m1)
    den = 1.0 + e2
    w1 = 1.0 / den
    w2 = e2 / den
    cost_ref[...] = m1 * w1 + m2 * w2
    gates_ref[...] = jnp.where(idx == a1, w1, 0.0) + jnp.where(idx == a2, w2, 0.0)
    win_ref[...] = jnp.concatenate([a1, a2], axis=-1)
    diff_ref[...] = diff
    aff_ref[...] = aff


def _router(h, mix, Wbridge, bb, conf_w, conf_b, Wd, Wa, cs):
    n = T // _TB3
    blk = pl.BlockSpec((_TB3, C), lambda i: (i, 0))
    wspec = pl.BlockSpec((C, C), lambda i: (0, 0))
    vec = pl.BlockSpec((1, C), lambda i: (0, 0))
    cE = pl.BlockSpec((C, E), lambda i: (0, 0))
    vE = pl.BlockSpec((1, E), lambda i: (0, 0))
    c1 = pl.BlockSpec((C, 1), lambda i: (0, 0))
    bE = pl.BlockSpec((_TB3, E), lambda i: (i, 0))
    b2c = pl.BlockSpec((_TB3, 2), lambda i: (i, 0))
    b1c = pl.BlockSpec((_TB3, 1), lambda i: (i, 0))
    return pl.pallas_call(
        _k3b,
        grid=(n,),
        in_specs=[blk, blk, wspec, wspec, vec, cE, vE, c1, cE, vE],
        out_specs=[blk, bE, b2c, b1c, b1c, bE],
        out_shape=[
            jax.ShapeDtypeStruct((T, C), jnp.float32),
            jax.ShapeDtypeStruct((T, E), jnp.float32),
            jax.ShapeDtypeStruct((T, 2), jnp.int32),
            jax.ShapeDtypeStruct((T, 1), jnp.float32),
            jax.ShapeDtypeStruct((T, 1), jnp.float32),
            jax.ShapeDtypeStruct((T, E), jnp.float32),
        ],
    )(h, mix, Wbridge[:C], Wbridge[C:], bb.reshape(1, C),
      conf_w.T, conf_b.reshape(1, E), Wd, Wa, cs.reshape(1, E))


def _k4a(h_ref, gates_ref, x1_ref, w1_ref, w2_ref, o_ref):
    e = pl.program_id(0)
    f = pl.program_id(1)

    @pl.when((e == 0) & (f == 0))
    def _():
        o_ref[...] = x1_ref[...]

    t = jnp.maximum(_dot(h_ref[...], w1_ref[0]), 0.0)
    ffn = _dot(t * t, w2_ref[0])
    g = gates_ref[...]
    sel = jax.lax.broadcasted_iota(jnp.int32, g.shape, 1) == e
    gate = jnp.sum(jnp.where(sel, g, 0.0), axis=-1, keepdims=True)
    o_ref[...] += gate * ffn


def _rwkv_experts(h, gates, x1, W1, W2):
    blk = pl.BlockSpec((T, C), lambda e, f: (0, 0))
    gspec = pl.BlockSpec((T, E), lambda e, f: (0, 0))
    w1spec = pl.BlockSpec((1, C, _FFC), lambda e, f: (e, 0, f))
    w2spec = pl.BlockSpec((1, _FFC, C), lambda e, f: (e, f, 0))
    return pl.pallas_call(
        _k4a,
        grid=(NR, FF // _FFC),
        in_specs=[blk, gspec, blk, w1spec, w2spec],
        out_specs=blk,
        out_shape=jax.ShapeDtypeStruct((T, C), jnp.float32),
    )(h, gates, x1, W1, W2)


def _k4b(h_ref, pre_ref, gates_ref, base_ref, wq_ref, wk_ref, wv_ref,
         wo_ref, o_ref):
    j = pl.program_id(0)
    c = pl.program_id(1)

    @pl.when((j == 0) & (c == 0))
    def _():
        o_ref[...] = base_ref[...]

    h = h_ref[...]
    q = _dot(h, wq_ref[0])
    kk = _dot(h, wk_ref[0])
    vv = _dot(h, wv_ref[0])
    z = q + pre_ref[...]
    act = z * jax.nn.sigmoid(z) * kk + vv
    eo = _dot(act, wo_ref[0])
    g = gates_ref[...]
    sel = jax.lax.broadcasted_iota(jnp.int32, g.shape, 1) == NR + j
    gate = jnp.sum(jnp.where(sel, g, 0.0), axis=-1, keepdims=True)
    o_ref[...] += gate * eo


def _trans_experts(h, pre, gates, base, Wq, Wk, Wv, Wo):
    blk = pl.BlockSpec((T, C), lambda j, c: (0, 0))
    gspec = pl.BlockSpec((T, E), lambda j, c: (0, 0))
    cspec = pl.BlockSpec((T, _CC), lambda j, c: (0, c))
    win = pl.BlockSpec((1, C, _CC), lambda j, c: (j, 0, c))
    wout = pl.BlockSpec((1, _CC, C), lambda j, c: (j, c, 0))
    return pl.pallas_call(
        _k4b,
        grid=(NT, C // _CC),
        in_specs=[blk, cspec, gspec, blk, win, win, win, wout],
        out_specs=blk,
        out_shape=jax.ShapeDtypeStruct((T, C), jnp.float32),
    )(h, pre, gates, base, Wq, Wk, Wv, Wo)


def kernel(x, v_first, capital_shares, ln1_g, ln1_b, ln2_g, ln2_b, Wr, Wk,
           Wv, Wo, Wbridge, bbridge, rwkv_W1, rwkv_W2, trans_Wq, trans_Wk,
           trans_Wv, trans_Wo, conf_w, conf_b, critic_Wd, critic_Wa, step,
           warmup_steps):
    x2 = x.reshape(T, C)
    x_ln = _ln(x, ln1_g, ln1_b).reshape(T, C)
    r, k, v = _qkv(x_ln, Wr, Wk, Wv)
    rh = r.reshape(T, H, HS).transpose(1, 0, 2)
    kh = k.reshape(T, H, HS).transpose(1, 0, 2)
    vh = v.reshape(T, H, HS).transpose(1, 0, 2)
    pu = _scores(rh, kh)
    z = jnp.sum(pu.reshape(1, H, T, T), axis=-1, keepdims=True).reshape(H, T, 1)
    mix = _pv(pu, z, vh).transpose(1, 0, 2).reshape(T, C)
    x1 = _resid(x2, mix, Wo)
    h = _ln(x1.reshape(1, T, C), ln2_g, ln2_b).reshape(T, C)
    pre, gates, winners, costs, diff, aff = _router(
        h, mix, Wbridge, bbridge, conf_w, conf_b, critic_Wd, critic_Wa,
        capital_shares)
    acc = _rwkv_experts(h, gates, x1, rwkv_W1, rwkv_W2)
    x_out = _trans_experts(h, pre, gates, acc, trans_Wq, trans_Wk, trans_Wv,
                           trans_Wo)
    return (x_out.reshape(1, T, C), v.reshape(1, T, C),
            winners.reshape(1, T, 2), costs.reshape(1, T),
            diff.reshape(1, T, 1), aff.reshape(1, T, E))


# speedup vs baseline: 1.0123x; 1.0123x over previous
import jax
import jax.numpy as jnp
from jax.experimental import pallas as pl

T, C = 2048, 1024
H, HS = 16, 64
NR, NT = 6, 2
E = NR + NT
FF = 2048

_TB1 = 512
_QB = 1024
_TB3 = 256
_TBE = 512
_FFC = 1024


def _ln(x, g, b):
    m = jnp.mean(x, axis=-1, keepdims=True)
    v = jnp.var(x, axis=-1, keepdims=True)
    return (x - m) / jnp.sqrt(v + 1e-5) * g + b


def _dot(a, b):
    return jnp.dot(a, b, preferred_element_type=jnp.float32)


def _k1(x_ref, wr_ref, wk_ref, wv_ref, r_ref, k_ref, v_ref):
    xln = x_ref[...]
    r_ref[...] = _dot(xln, wr_ref[...])
    k_ref[...] = _dot(xln, wk_ref[...])
    v_ref[...] = _dot(xln, wv_ref[...])


def _qkv(x_ln, Wr, Wk, Wv):
    n = T // _TB1
    blk = pl.BlockSpec((_TB1, C), lambda i: (i, 0))
    wspec = pl.BlockSpec((C, C), lambda i: (0, 0))
    return pl.pallas_call(
        _k1,
        grid=(n,),
        in_specs=[blk, wspec, wspec, wspec],
        out_specs=[blk, blk, blk],
        out_shape=[jax.ShapeDtypeStruct((T, C), jnp.float32)] * 3,
    )(x_ln, Wr, Wk, Wv)


def _k2a(r_ref, k_ref, o_ref):
    qi = pl.program_id(1)
    s = jax.lax.dot_general(
        r_ref[0], k_ref[0], (((1,), (1,)), ((), ())),
        preferred_element_type=jnp.float32) / 8.0
    rows = qi * _QB + jax.lax.broadcasted_iota(jnp.int32, (_QB, T), 0)
    cols = jax.lax.broadcasted_iota(jnp.int32, (_QB, T), 1)
    s = jnp.where(cols <= rows, s, -1e9)
    m = jnp.max(s, axis=-1, keepdims=True)
    o_ref[0] = jnp.exp(s - m)


def _scores(rh, kh):
    qspec = pl.BlockSpec((1, _QB, HS), lambda h, q: (h, q, 0))
    kspec = pl.BlockSpec((1, T, HS), lambda h, q: (h, 0, 0))
    return pl.pallas_call(
        _k2a,
        grid=(H, T // _QB),
        in_specs=[qspec, kspec],
        out_specs=pl.BlockSpec((1, _QB, T), lambda h, q: (h, q, 0)),
        out_shape=jax.ShapeDtypeStruct((H, T, T), jnp.float32),
    )(rh, kh)


def _k2b(pu_ref, z_ref, v_ref, o_ref):
    o_ref[0] = _dot(pu_ref[0] / z_ref[0], v_ref[0])


def _pv(pu, z, vh):
    return pl.pallas_call(
        _k2b,
        grid=(H, T // _QB),
        in_specs=[pl.BlockSpec((1, _QB, T), lambda h, q: (h, q, 0)),
                  pl.BlockSpec((1, _QB, 1), lambda h, q: (h, q, 0)),
                  pl.BlockSpec((1, T, HS), lambda h, q: (h, 0, 0))],
        out_specs=pl.BlockSpec((1, _QB, HS), lambda h, q: (h, q, 0)),
        out_shape=jax.ShapeDtypeStruct((H, T, HS), jnp.float32),
    )(pu, z, vh)


def _k3a(x_ref, mix_ref, wo_ref, o_ref):
    o_ref[...] = x_ref[...] + _dot(mix_ref[...], wo_ref[...])


def _resid(x, mix, Wo):
    n = T // _TB3
    blk = pl.BlockSpec((_TB3, C), lambda i: (i, 0))
    wspec = pl.BlockSpec((C, C), lambda i: (0, 0))
    return pl.pallas_call(
        _k3a,
        grid=(n,),
        in_specs=[blk, blk, wspec],
        out_specs=blk,
        out_shape=jax.ShapeDtypeStruct((T, C), jnp.float32),
    )(x, mix, Wo)


def _k3b(h_ref, mix_ref, wbh_ref, wbs_ref, bb_ref,
         cw_ref, cb_ref, wd_ref, wa_ref, cs_ref,
         pre_ref, gates_ref, win_ref, cost_ref, diff_ref, aff_ref):
    h = h_ref[...]
    mix = mix_ref[...]
    pre_ref[...] = jnp.maximum(
        _dot(h, wbh_ref[...]) + _dot(mix, wbs_ref[...]) + bb_ref[...], 0.0)
    conf = jax.nn.sigmoid(_dot(h, cw_ref[...]) + cb_ref[...])
    diff = jax.nn.sigmoid(_dot(h, wd_ref[...]))
    aff = _dot(h, wa_ref[...])
    bids = conf * cs_ref[...] * diff + 0.1 * aff
    idx = jax.lax.broadcasted_iota(jnp.int32, bids.shape, 1)
    m1 = jnp.max(bids, axis=-1, keepdims=True)
    a1 = jnp.argmax(bids, axis=-1, keepdims=True).astype(jnp.int32)
    masked = jnp.where(idx == a1, -jnp.inf, bids)
    m2 = jnp.max(masked, axis=-1, keepdims=True)
    a2 = jnp.argmax(masked, axis=-1, keepdims=True).astype(jnp.int32)
    e2 = jnp.exp(m2 - m1)
    den = 1.0 + e2
    w1 = 1.0 / den
    w2 = e2 / den
    cost_ref[...] = m1 * w1 + m2 * w2
    gates_ref[...] = jnp.where(idx == a1, w1, 0.0) + jnp.where(idx == a2, w2, 0.0)
    win_ref[...] = jnp.concatenate([a1, a2], axis=-1)
    diff_ref[...] = diff
    aff_ref[...] = aff


def _router(h, mix, Wbridge, bb, conf_w, conf_b, Wd, Wa, cs):
    n = T // _TB3
    blk = pl.BlockSpec((_TB3, C), lambda i: (i, 0))
    wspec = pl.BlockSpec((C, C), lambda i: (0, 0))
    vec = pl.BlockSpec((1, C), lambda i: (0, 0))
    cE = pl.BlockSpec((C, E), lambda i: (0, 0))
    vE = pl.BlockSpec((1, E), lambda i: (0, 0))
    c1 = pl.BlockSpec((C, 1), lambda i: (0, 0))
    bE = pl.BlockSpec((_TB3, E), lambda i: (i, 0))
    b2c = pl.BlockSpec((_TB3, 2), lambda i: (i, 0))
    b1c = pl.BlockSpec((_TB3, 1), lambda i: (i, 0))
    return pl.pallas_call(
        _k3b,
        grid=(n,),
        in_specs=[blk, blk, wspec, wspec, vec, cE, vE, c1, cE, vE],
        out_specs=[blk, bE, b2c, b1c, b1c, bE],
        out_shape=[
            jax.ShapeDtypeStruct((T, C), jnp.float32),
            jax.ShapeDtypeStruct((T, E), jnp.float32),
            jax.ShapeDtypeStruct((T, 2), jnp.int32),
            jax.ShapeDtypeStruct((T, 1), jnp.float32),
            jax.ShapeDtypeStruct((T, 1), jnp.float32),
            jax.ShapeDtypeStruct((T, E), jnp.float32),
        ],
    )(h, mix, Wbridge[:C], Wbridge[C:], bb.reshape(1, C),
      conf_w.T, conf_b.reshape(1, E), Wd, Wa, cs.reshape(1, E))


def _k4a(h_ref, gates_ref, x1_ref, w1_ref, w2_ref, o_ref):
    e = pl.program_id(0)
    f = pl.program_id(1)

    @pl.when((e == 0) & (f == 0))
    def _():
        o_ref[...] = x1_ref[...]

    t = jnp.maximum(_dot(h_ref[...], w1_ref[0]), 0.0)
    ffn = _dot(t * t, w2_ref[0])
    g = gates_ref[...]
    sel = jax.lax.broadcasted_iota(jnp.int32, g.shape, 1) == e
    gate = jnp.sum(jnp.where(sel, g, 0.0), axis=-1, keepdims=True)
    o_ref[...] += gate * ffn


def _rwkv_experts(h, gates, x1, W1, W2):
    blk = pl.BlockSpec((T, C), lambda e, f: (0, 0))
    gspec = pl.BlockSpec((T, E), lambda e, f: (0, 0))
    w1spec = pl.BlockSpec((1, C, _FFC), lambda e, f: (e, 0, f))
    w2spec = pl.BlockSpec((1, _FFC, C), lambda e, f: (e, f, 0))
    return pl.pallas_call(
        _k4a,
        grid=(NR, FF // _FFC),
        in_specs=[blk, gspec, blk, w1spec, w2spec],
        out_specs=blk,
        out_shape=jax.ShapeDtypeStruct((T, C), jnp.float32),
    )(h, gates, x1, W1, W2)


def _k4b(h_ref, pre_ref, gates_ref, base_ref, wq_ref, wk_ref, wv_ref,
         wo_ref, o_ref):
    j = pl.program_id(1)

    @pl.when(j == 0)
    def _():
        o_ref[...] = base_ref[...]

    h = h_ref[...]
    q = _dot(h, wq_ref[0])
    kk = _dot(h, wk_ref[0])
    vv = _dot(h, wv_ref[0])
    z = q + pre_ref[...]
    act = z * jax.nn.sigmoid(z) * kk + vv
    eo = _dot(act, wo_ref[0])
    g = gates_ref[...]
    sel = jax.lax.broadcasted_iota(jnp.int32, g.shape, 1) == NR + j
    gate = jnp.sum(jnp.where(sel, g, 0.0), axis=-1, keepdims=True)
    o_ref[...] += gate * eo


def _trans_experts(h, pre, gates, base, Wq, Wk, Wv, Wo):
    n = T // _TBE
    blk = pl.BlockSpec((_TBE, C), lambda t, j: (t, 0))
    gspec = pl.BlockSpec((_TBE, E), lambda t, j: (t, 0))
    wspec = pl.BlockSpec((1, C, C), lambda t, j: (j, 0, 0))
    return pl.pallas_call(
        _k4b,
        grid=(n, NT),
        in_specs=[blk, blk, gspec, blk, wspec, wspec, wspec, wspec],
        out_specs=blk,
        out_shape=jax.ShapeDtypeStruct((T, C), jnp.float32),
    )(h, pre, gates, base, Wq, Wk, Wv, Wo)


def kernel(x, v_first, capital_shares, ln1_g, ln1_b, ln2_g, ln2_b, Wr, Wk,
           Wv, Wo, Wbridge, bbridge, rwkv_W1, rwkv_W2, trans_Wq, trans_Wk,
           trans_Wv, trans_Wo, conf_w, conf_b, critic_Wd, critic_Wa, step,
           warmup_steps):
    x2 = x.reshape(T, C)
    x_ln = _ln(x, ln1_g, ln1_b).reshape(T, C)
    r, k, v = _qkv(x_ln, Wr, Wk, Wv)
    rh = r.reshape(T, H, HS).transpose(1, 0, 2)
    kh = k.reshape(T, H, HS).transpose(1, 0, 2)
    vh = v.reshape(T, H, HS).transpose(1, 0, 2)
    pu = _scores(rh, kh)
    z = jnp.sum(pu.reshape(1, H, T, T), axis=-1, keepdims=True).reshape(H, T, 1)
    mix = _pv(pu, z, vh).transpose(1, 0, 2).reshape(T, C)
    x1 = _resid(x2, mix, Wo)
    h = _ln(x1.reshape(1, T, C), ln2_g, ln2_b).reshape(T, C)
    pre, gates, winners, costs, diff, aff = _router(
        h, mix, Wbridge, bbridge, conf_w, conf_b, critic_Wd, critic_Wa,
        capital_shares)
    acc = _rwkv_experts(h, gates, x1, rwkv_W1, rwkv_W2)
    x_out = _trans_experts(h, pre, gates, acc, trans_Wq, trans_Wk, trans_Wv,
                           trans_Wo)
    return (x_out.reshape(1, T, C), v.reshape(1, T, C),
            winners.reshape(1, T, 2), costs.reshape(1, T),
            diff.reshape(1, T, 1), aff.reshape(1, T, E))


# speedup vs baseline: 1.0567x; 1.0438x over previous
import jax
import jax.numpy as jnp
from jax.experimental import pallas as pl

T, C = 2048, 1024
H, HS = 16, 64
NR, NT = 6, 2
E = NR + NT
FF = 2048

_TB1 = 512
_QB = 1024
_TB3 = 256
_TBE = 512
_FFC = 1024


def _ln(x, g, b):
    m = jnp.mean(x, axis=-1, keepdims=True)
    v = jnp.var(x, axis=-1, keepdims=True)
    return (x - m) / jnp.sqrt(v + 1e-5) * g + b


def _dot(a, b):
    return jnp.dot(a, b, preferred_element_type=jnp.float32)


def _k1(x_ref, wr_ref, wk_ref, wv_ref, r_ref, k_ref, v_ref):
    xln = x_ref[...]
    r_ref[...] = _dot(xln, wr_ref[...])
    k_ref[...] = _dot(xln, wk_ref[...])
    v_ref[...] = _dot(xln, wv_ref[...])


def _qkv(x_ln, Wr, Wk, Wv):
    n = T // _TB1
    blk = pl.BlockSpec((_TB1, C), lambda i: (i, 0))
    wspec = pl.BlockSpec((C, C), lambda i: (0, 0))
    return pl.pallas_call(
        _k1,
        grid=(n,),
        in_specs=[blk, wspec, wspec, wspec],
        out_specs=[blk, blk, blk],
        out_shape=[jax.ShapeDtypeStruct((T, C), jnp.float32)] * 3,
    )(x_ln, Wr, Wk, Wv)


def _k2a(r_ref, k_ref, v_ref, pu_ref, ou_ref):
    qi = pl.program_id(1)
    s = jax.lax.dot_general(
        r_ref[0], k_ref[0], (((1,), (1,)), ((), ())),
        preferred_element_type=jnp.float32) / 8.0
    rows = qi * _QB + jax.lax.broadcasted_iota(jnp.int32, (_QB, T), 0)
    cols = jax.lax.broadcasted_iota(jnp.int32, (_QB, T), 1)
    s = jnp.where(cols <= rows, s, -1e9)
    m = jnp.max(s, axis=-1, keepdims=True)
    pu = jnp.exp(s - m)
    pu_ref[0] = pu
    ou_ref[0] = _dot(pu, v_ref[0])


def _attention(rh, kh, vh):
    qspec = pl.BlockSpec((1, _QB, HS), lambda h, q: (h, q, 0))
    kspec = pl.BlockSpec((1, T, HS), lambda h, q: (h, 0, 0))
    return pl.pallas_call(
        _k2a,
        grid=(H, T // _QB),
        in_specs=[qspec, kspec, kspec],
        out_specs=[pl.BlockSpec((1, _QB, T), lambda h, q: (h, q, 0)), qspec],
        out_shape=[jax.ShapeDtypeStruct((H, T, T), jnp.float32),
                   jax.ShapeDtypeStruct((H, T, HS), jnp.float32)],
    )(rh, kh, vh)


def _k3a(x_ref, mix_ref, wo_ref, o_ref):
    o_ref[...] = x_ref[...] + _dot(mix_ref[...], wo_ref[...])


def _resid(x, mix, Wo):
    n = T // _TB3
    blk = pl.BlockSpec((_TB3, C), lambda i: (i, 0))
    wspec = pl.BlockSpec((C, C), lambda i: (0, 0))
    return pl.pallas_call(
        _k3a,
        grid=(n,),
        in_specs=[blk, blk, wspec],
        out_specs=blk,
        out_shape=jax.ShapeDtypeStruct((T, C), jnp.float32),
    )(x, mix, Wo)


def _k3b(h_ref, mix_ref, wbh_ref, wbs_ref, bb_ref,
         cw_ref, cb_ref, wd_ref, wa_ref, cs_ref,
         pre_ref, gates_ref, win_ref, cost_ref, diff_ref, aff_ref):
    h = h_ref[...]
    mix = mix_ref[...]
    pre_ref[...] = jnp.maximum(
        _dot(h, wbh_ref[...]) + _dot(mix, wbs_ref[...]) + bb_ref[...], 0.0)
    conf = jax.nn.sigmoid(_dot(h, cw_ref[...]) + cb_ref[...])
    diff = jax.nn.sigmoid(_dot(h, wd_ref[...]))
    aff = _dot(h, wa_ref[...])
    bids = conf * cs_ref[...] * diff + 0.1 * aff
    idx = jax.lax.broadcasted_iota(jnp.int32, bids.shape, 1)
    m1 = jnp.max(bids, axis=-1, keepdims=True)
    a1 = jnp.argmax(bids, axis=-1, keepdims=True).astype(jnp.int32)
    masked = jnp.where(idx == a1, -jnp.inf, bids)
    m2 = jnp.max(masked, axis=-1, keepdims=True)
    a2 = jnp.argmax(masked, axis=-1, keepdims=True).astype(jnp.int32)
    e2 = jnp.exp(m2 - m1)
    den = 1.0 + e2
    w1 = 1.0 / den
    w2 = e2 / den
    cost_ref[...] = m1 * w1 + m2 * w2
    gates_ref[...] = jnp.where(idx == a1, w1, 0.0) + jnp.where(idx == a2, w2, 0.0)
    win_ref[...] = jnp.concatenate([a1, a2], axis=-1)
    diff_ref[...] = diff
    aff_ref[...] = aff


def _router(h, mix, Wbridge, bb, conf_w, conf_b, Wd, Wa, cs):
    n = T // _TB3
    blk = pl.BlockSpec((_TB3, C), lambda i: (i, 0))
    wspec = pl.BlockSpec((C, C), lambda i: (0, 0))
    vec = pl.BlockSpec((1, C), lambda i: (0, 0))
    cE = pl.BlockSpec((C, E), lambda i: (0, 0))
    vE = pl.BlockSpec((1, E), lambda i: (0, 0))
    c1 = pl.BlockSpec((C, 1), lambda i: (0, 0))
    bE = pl.BlockSpec((_TB3, E), lambda i: (i, 0))
    b2c = pl.BlockSpec((_TB3, 2), lambda i: (i, 0))
    b1c = pl.BlockSpec((_TB3, 1), lambda i: (i, 0))
    return pl.pallas_call(
        _k3b,
        grid=(n,),
        in_specs=[blk, blk, wspec, wspec, vec, cE, vE, c1, cE, vE],
        out_specs=[blk, bE, b2c, b1c, b1c, bE],
        out_shape=[
            jax.ShapeDtypeStruct((T, C), jnp.float32),
            jax.ShapeDtypeStruct((T, E), jnp.float32),
            jax.ShapeDtypeStruct((T, 2), jnp.int32),
            jax.ShapeDtypeStruct((T, 1), jnp.float32),
            jax.ShapeDtypeStruct((T, 1), jnp.float32),
            jax.ShapeDtypeStruct((T, E), jnp.float32),
        ],
    )(h, mix, Wbridge[:C], Wbridge[C:], bb.reshape(1, C),
      conf_w.T, conf_b.reshape(1, E), Wd, Wa, cs.reshape(1, E))


def _k4a(h_ref, gates_ref, x1_ref, w1_ref, w2_ref, o_ref):
    e = pl.program_id(0)
    f = pl.program_id(1)

    @pl.when((e == 0) & (f == 0))
    def _():
        o_ref[...] = x1_ref[...]

    t = jnp.maximum(_dot(h_ref[...], w1_ref[0]), 0.0)
    ffn = _dot(t * t, w2_ref[0])
    g = gates_ref[...]
    sel = jax.lax.broadcasted_iota(jnp.int32, g.shape, 1) == e
    gate = jnp.sum(jnp.where(sel, g, 0.0), axis=-1, keepdims=True)
    o_ref[...] += gate * ffn


def _rwkv_experts(h, gates, x1, W1, W2):
    blk = pl.BlockSpec((T, C), lambda e, f: (0, 0))
    gspec = pl.BlockSpec((T, E), lambda e, f: (0, 0))
    w1spec = pl.BlockSpec((1, C, _FFC), lambda e, f: (e, 0, f))
    w2spec = pl.BlockSpec((1, _FFC, C), lambda e, f: (e, f, 0))
    return pl.pallas_call(
        _k4a,
        grid=(NR, FF // _FFC),
        in_specs=[blk, gspec, blk, w1spec, w2spec],
        out_specs=blk,
        out_shape=jax.ShapeDtypeStruct((T, C), jnp.float32),
    )(h, gates, x1, W1, W2)


def _k4b(h_ref, pre_ref, gates_ref, base_ref, wq_ref, wk_ref, wv_ref,
         wo_ref, o_ref):
    j = pl.program_id(1)

    @pl.when(j == 0)
    def _():
        o_ref[...] = base_ref[...]

    h = h_ref[...]
    q = _dot(h, wq_ref[0])
    kk = _dot(h, wk_ref[0])
    vv = _dot(h, wv_ref[0])
    z = q + pre_ref[...]
    act = z * jax.nn.sigmoid(z) * kk + vv
    eo = _dot(act, wo_ref[0])
    g = gates_ref[...]
    sel = jax.lax.broadcasted_iota(jnp.int32, g.shape, 1) == NR + j
    gate = jnp.sum(jnp.where(sel, g, 0.0), axis=-1, keepdims=True)
    o_ref[...] += gate * eo


def _trans_experts(h, pre, gates, base, Wq, Wk, Wv, Wo):
    n = T // _TBE
    blk = pl.BlockSpec((_TBE, C), lambda t, j: (t, 0))
    gspec = pl.BlockSpec((_TBE, E), lambda t, j: (t, 0))
    wspec = pl.BlockSpec((1, C, C), lambda t, j: (j, 0, 0))
    return pl.pallas_call(
        _k4b,
        grid=(n, NT),
        in_specs=[blk, blk, gspec, blk, wspec, wspec, wspec, wspec],
        out_specs=blk,
        out_shape=jax.ShapeDtypeStruct((T, C), jnp.float32),
    )(h, pre, gates, base, Wq, Wk, Wv, Wo)


def kernel(x, v_first, capital_shares, ln1_g, ln1_b, ln2_g, ln2_b, Wr, Wk,
           Wv, Wo, Wbridge, bbridge, rwkv_W1, rwkv_W2, trans_Wq, trans_Wk,
           trans_Wv, trans_Wo, conf_w, conf_b, critic_Wd, critic_Wa, step,
           warmup_steps):
    x2 = x.reshape(T, C)
    x_ln = _ln(x, ln1_g, ln1_b).reshape(T, C)
    r, k, v = _qkv(x_ln, Wr, Wk, Wv)
    rh = r.reshape(T, H, HS).transpose(1, 0, 2)
    kh = k.reshape(T, H, HS).transpose(1, 0, 2)
    vh = v.reshape(T, H, HS).transpose(1, 0, 2)
    pu, ou = _attention(rh, kh, vh)
    z = jnp.sum(pu.reshape(1, H, T, T), axis=-1, keepdims=True).reshape(H, T, 1)
    mix = (ou / z).transpose(1, 0, 2).reshape(T, C)
    x1 = _resid(x2, mix, Wo)
    h = _ln(x1.reshape(1, T, C), ln2_g, ln2_b).reshape(T, C)
    pre, gates, winners, costs, diff, aff = _router(
        h, mix, Wbridge, bbridge, conf_w, conf_b, critic_Wd, critic_Wa,
        capital_shares)
    acc = _rwkv_experts(h, gates, x1, rwkv_W1, rwkv_W2)
    x_out = _trans_experts(h, pre, gates, acc, trans_Wq, trans_Wk, trans_Wv,
                           trans_Wo)
    return (x_out.reshape(1, T, C), v.reshape(1, T, C),
            winners.reshape(1, T, 2), costs.reshape(1, T),
            diff.reshape(1, T, 1), aff.reshape(1, T, E))
